# baseline (speedup 1.0000x reference)
import jax
import jax.numpy as jnp
from jax import lax
from jax.experimental import pallas as pl
from jax.experimental.pallas import tpu as pltpu
from jax.experimental.pallas import tpu_sc as plsc

NUM_ENTITIES = 1000000
NUM_RELATIONS = 1000
EMBED_DIM = 64
BATCH = 16384

NC = 2
NS = 16
NW = NC * NS
B_PER_W = BATCH // NW
CHUNK = 128
NCHUNK = B_PER_W // CHUNK
L = 16
D2 = 2 * EMBED_DIM


def _body(head_r, rel_r, tail_r, ent2, rel2, out_hbm,
          idx_h, idx_r, idx_t, g_h, g_t, g_r, out_v, sem):
    wid = lax.axis_index("s") * NC + lax.axis_index("c")
    iota = lax.iota(jnp.int32, L)

    def chunk_body(ci, carry):
        base = wid * B_PER_W + ci * CHUNK
        pltpu.sync_copy(head_r.at[pl.ds(base, CHUNK)], idx_h)
        pltpu.sync_copy(rel_r.at[pl.ds(base, CHUNK)], idx_r)
        pltpu.sync_copy(tail_r.at[pl.ds(base, CHUNK)], idx_t)
        cps = [
            pltpu.async_copy(ent2.at[idx_h], g_h, sem),
            pltpu.async_copy(ent2.at[idx_t], g_t, sem),
            pltpu.async_copy(rel2.at[idx_r], g_r, sem),
        ]
        for cp in cps:
            cp.wait()

        def group_body(g, carry2):
            tot = jnp.zeros((L,), jnp.float32)
            for row in range(L):
                c = g * L + row
                acc = jnp.zeros((L,), jnp.float32)
                for j in range(EMBED_DIM // L):
                    slr = pl.ds(j * L, L)
                    sli = pl.ds(EMBED_DIM + j * L, L)
                    hr = g_h[c, slr]
                    hi = g_h[c, sli]
                    tr = g_t[c, slr]
                    ti = g_t[c, sli]
                    rr = g_r[c, slr]
                    ri = g_r[c, sli]
                    acc = acc + rr * (hr * tr + hi * ti) + ri * (hr * ti - hi * tr)
                s = lax.reduce_sum_p.bind(acc, axes=(0,))
                tot = jnp.where(iota == row, s, tot)
            out_v[pl.ds(g * L, L)] = tot
            return carry2

        lax.fori_loop(0, CHUNK // L, group_body, 0)
        pltpu.sync_copy(out_v, out_hbm.at[pl.ds(base, CHUNK)])
        return carry

    lax.fori_loop(0, NCHUNK, chunk_body, 0)


def kernel(head, relation, tail, ent_real, ent_imag, rel_real, rel_imag):
    ent2 = jnp.concatenate([ent_real, ent_imag], axis=1)
    rel2 = jnp.concatenate([rel_real, rel_imag], axis=1)
    mesh = plsc.VectorSubcoreMesh(core_axis_name="c", subcore_axis_name="s")
    f = pl.kernel(
        _body,
        mesh=mesh,
        compiler_params=pltpu.CompilerParams(
            needs_layout_passes=False, use_tc_tiling_on_sc=True),
        out_type=jax.ShapeDtypeStruct((BATCH,), jnp.float32),
        scratch_types=[
            pltpu.VMEM((CHUNK,), jnp.int32),
            pltpu.VMEM((CHUNK,), jnp.int32),
            pltpu.VMEM((CHUNK,), jnp.int32),
            pltpu.VMEM((CHUNK, D2), jnp.float32),
            pltpu.VMEM((CHUNK, D2), jnp.float32),
            pltpu.VMEM((CHUNK, D2), jnp.float32),
            pltpu.VMEM((CHUNK,), jnp.float32),
            pltpu.SemaphoreType.DMA,
        ],
    )
    return f(head, relation, tail, ent2, rel2)
